# baseline (speedup 1.0000x reference)
import functools

import jax
import jax.numpy as jnp
from jax import lax
from jax.experimental import pallas as pl
from jax.experimental.pallas import tpu as pltpu
from jax.experimental.pallas import tpu_sc as plsc

T = 2048
D = 2048
E = 8
K = 2
H = 1408
H2 = 2 * H

B = 128
NB = (K * T) // B + E
S = K * T
S_PAD = NB * B

NC, NS = 2, 16
NW = NC * NS


def _gate_body(x_ref, wg_ref, w_out_ref, e_out_ref):
    sT = lax.dot_general(wg_ref[...], x_ref[...], (((1,), (1,)), ((), ())),
                         preferred_element_type=jnp.float32)
    bt = sT.shape[1]
    m = jnp.max(sT, axis=0, keepdims=True)
    p = jnp.exp(sT - m)
    denom = jnp.sum(p, axis=0, keepdims=True)
    rows = lax.broadcasted_iota(jnp.int32, (E, bt), 0)
    p1 = jnp.max(p, axis=0, keepdims=True)
    e1 = jnp.min(jnp.where(p == p1, rows, E), axis=0, keepdims=True)
    p_m = jnp.where(rows == e1, -jnp.inf, p)
    p2 = jnp.max(p_m, axis=0, keepdims=True)
    e2 = jnp.min(jnp.where(p_m == p2, rows, E), axis=0, keepdims=True)
    w_out_ref[...] = jnp.concatenate([p1, p2], axis=0) / denom
    e_out_ref[...] = jnp.concatenate([e1, e2], axis=0)


def _gate(x, w_gate):
    BT = 512
    return pl.pallas_call(
        _gate_body,
        grid=(T // BT,),
        in_specs=[
            pl.BlockSpec((BT, D), lambda i: (i, 0)),
            pl.BlockSpec((E, D), lambda i: (0, 0)),
        ],
        out_specs=[
            pl.BlockSpec((K, BT), lambda i: (0, i)),
            pl.BlockSpec((K, BT), lambda i: (0, i)),
        ],
        out_shape=[
            jax.ShapeDtypeStruct((K, T), jnp.float32),
            jax.ShapeDtypeStruct((K, T), jnp.int32),
        ],
    )(x, w_gate)


def _routing(e_top, w_top):
    eflat = e_top.reshape(S)
    wflat = w_top.reshape(S)
    order = jnp.argsort(eflat, stable=True)
    sorted_e = eflat[order]
    counts = jnp.bincount(eflat, length=E)
    cum = jnp.concatenate([jnp.zeros(1, jnp.int32),
                           jnp.cumsum(counts)]).astype(jnp.int32)
    nblk = (counts + B - 1) // B
    bcum = jnp.concatenate([jnp.zeros(1, jnp.int32),
                            jnp.cumsum(nblk)]).astype(jnp.int32)
    pad_off = bcum * B
    barange = jnp.arange(NB, dtype=jnp.int32)
    be = jnp.clip(jnp.searchsorted(bcum[1:], barange, side='right'),
                  0, E - 1).astype(jnp.int32)
    nvalid = jnp.clip(counts[be] - (barange - bcum[be]) * B, 0, B)
    j = jnp.arange(S, dtype=jnp.int32)
    padpos = pad_off[sorted_e] + (j - cum[sorted_e])
    tid_pad = jnp.zeros(S_PAD, jnp.int32).at[padpos].set(
        (order % T).astype(jnp.int32))
    ws_pad = jnp.zeros(S_PAD, jnp.float32).at[padpos].set(wflat[order])
    posflat = jnp.zeros(S, jnp.int32).at[order].set(padpos)
    return be, nvalid.astype(jnp.int32), tid_pad, ws_pad, posflat


def _sc_gather(x, tid_pad):
    CH = 32
    RW = S_PAD // NW
    mesh = plsc.VectorSubcoreMesh(core_axis_name="c", subcore_axis_name="s")

    @functools.partial(
        pl.kernel,
        out_type=jax.ShapeDtypeStruct((S_PAD, D), jnp.float32),
        mesh=mesh,
        scratch_types=[
            pltpu.VMEM((CH,), jnp.int32),
            pltpu.VMEM((CH, D), jnp.float32),
            pltpu.SemaphoreType.DMA,
        ],
    )
    def k(x_hbm, tid_hbm, xs_hbm, idx_v, rows_v, sem):
        wid = lax.axis_index("s") * NC + lax.axis_index("c")
        base = wid * RW
        for c in range(RW // CH):
            off = base + c * CH
            pltpu.sync_copy(tid_hbm.at[pl.ds(off, CH)], idx_v)
            pltpu.async_copy(x_hbm.at[idx_v], rows_v, sem).wait()
            pltpu.sync_copy(rows_v, xs_hbm.at[pl.ds(off, CH)])

    return k(x, tid_pad)


def _fc1_body(be_ref, nv_ref, xs_ref, w1_ref, h_ref):
    b = pl.program_id(0)

    @pl.when(nv_ref[b] > 0)
    def _():
        y = lax.dot_general(xs_ref[...], w1_ref[0], (((1,), (1,)), ((), ())),
                            preferred_element_type=jnp.float32)
        g = y[:, H:]
        h_ref[...] = y[:, :H] * (g * jax.nn.sigmoid(g))


def _experts_fc1(xs, e_w1, be, nvalid):
    grid_spec = pltpu.PrefetchScalarGridSpec(
        num_scalar_prefetch=2,
        grid=(NB,),
        in_specs=[
            pl.BlockSpec((B, D), lambda b, be, nv: (b, 0)),
            pl.BlockSpec((1, H2, D), lambda b, be, nv: (be[b], 0, 0)),
        ],
        out_specs=pl.BlockSpec((B, H), lambda b, be, nv: (b, 0)),
    )
    return pl.pallas_call(
        _fc1_body,
        grid_spec=grid_spec,
        out_shape=jax.ShapeDtypeStruct((S_PAD, H), jnp.float32),
        compiler_params=pltpu.CompilerParams(
            dimension_semantics=("arbitrary",)),
    )(be, nvalid, xs, e_w1)


def _fc2_body(be_ref, nv_ref, h_ref, ws_ref, w2_ref, out_ref):
    b = pl.program_id(0)

    @pl.when(nv_ref[b] > 0)
    def _():
        z = lax.dot_general(h_ref[...], w2_ref[0], (((1,), (1,)), ((), ())),
                            preferred_element_type=jnp.float32)
        out_ref[...] = z * ws_ref[...]


def _experts_fc2(h_all, ws_col, e_w2, be, nvalid):
    grid_spec = pltpu.PrefetchScalarGridSpec(
        num_scalar_prefetch=2,
        grid=(NB,),
        in_specs=[
            pl.BlockSpec((B, H), lambda b, be, nv: (b, 0)),
            pl.BlockSpec((B, 1), lambda b, be, nv: (b, 0)),
            pl.BlockSpec((1, D, H), lambda b, be, nv: (be[b], 0, 0)),
        ],
        out_specs=pl.BlockSpec((B, D), lambda b, be, nv: (b, 0)),
    )
    return pl.pallas_call(
        _fc2_body,
        grid_spec=grid_spec,
        out_shape=jax.ShapeDtypeStruct((S_PAD, D), jnp.float32),
        compiler_params=pltpu.CompilerParams(
            dimension_semantics=("arbitrary",)),
    )(be, nvalid, h_all, ws_col, e_w2)


def _sfc1_body(x_ref, w1_ref, h_ref):
    y = lax.dot_general(x_ref[...], w1_ref[...], (((1,), (1,)), ((), ())),
                        preferred_element_type=jnp.float32)
    g = y[:, H:]
    h_ref[...] = y[:, :H] * (g * jax.nn.sigmoid(g))


def _sfc2_body(h_ref, w2_ref, out_ref):
    out_ref[...] = lax.dot_general(h_ref[...], w2_ref[...],
                                   (((1,), (1,)), ((), ())),
                                   preferred_element_type=jnp.float32)


def _shared(x, s_w1, s_w2):
    BS = 256
    h_all = pl.pallas_call(
        _sfc1_body,
        grid=(T // BS,),
        in_specs=[
            pl.BlockSpec((BS, D), lambda i: (i, 0)),
            pl.BlockSpec((H2, D), lambda i: (0, 0)),
        ],
        out_specs=pl.BlockSpec((BS, H), lambda i: (i, 0)),
        out_shape=jax.ShapeDtypeStruct((T, H), jnp.float32),
    )(x, s_w1)
    return pl.pallas_call(
        _sfc2_body,
        grid=(T // BS,),
        in_specs=[
            pl.BlockSpec((BS, H), lambda i: (i, 0)),
            pl.BlockSpec((D, H), lambda i: (0, 0)),
        ],
        out_specs=pl.BlockSpec((BS, D), lambda i: (i, 0)),
        out_shape=jax.ShapeDtypeStruct((T, D), jnp.float32),
    )(h_all, s_w2)


def _sc_combine(shared_out, slot_out, pos2):
    CH = 16
    TW = T // NW
    mesh = plsc.VectorSubcoreMesh(core_axis_name="c", subcore_axis_name="s")

    @functools.partial(
        pl.kernel,
        out_type=jax.ShapeDtypeStruct((T, D), jnp.float32),
        mesh=mesh,
        scratch_types=[
            pltpu.VMEM((CH,), jnp.int32),
            pltpu.VMEM((CH, D), jnp.float32),
            pltpu.VMEM((CH, D), jnp.float32),
            pltpu.VMEM((CH, D), jnp.float32),
            pltpu.SemaphoreType.DMA,
        ],
    )
    def k(sh_hbm, sl_hbm, p2_hbm, out_hbm, idx_v, acc_v, r0_v, r1_v, sem):
        wid = lax.axis_index("s") * NC + lax.axis_index("c")
        base = wid * TW
        rbufs = (r0_v, r1_v)
        for c in range(TW // CH):
            off = base + c * CH
            pltpu.sync_copy(sh_hbm.at[pl.ds(off, CH)], acc_v)
            for j in range(K):
                pltpu.sync_copy(p2_hbm.at[pl.ds(j * T + off, CH)], idx_v)
                pltpu.async_copy(sl_hbm.at[idx_v], rbufs[j], sem).wait()

            def row_add(r, _):
                def col_add(ci, _):
                    sl = pl.ds(ci * 16, 16)
                    acc_v[r, sl] = (acc_v[r, sl] + r0_v[r, sl]
                                    + r1_v[r, sl])
                    return 0
                return lax.fori_loop(0, D // 16, col_add, 0)

            lax.fori_loop(0, CH, row_add, 0)
            pltpu.sync_copy(acc_v, out_hbm.at[pl.ds(off, CH)])

    return k(shared_out, slot_out, pos2)


def kernel(x, w_gate, e_w1, e_w2, s_w1, s_w2):
    w_top, e_top = _gate(x, w_gate)
    be, nvalid, tid_pad, ws_pad, pos2 = _routing(e_top, w_top)

    xs = _sc_gather(x, tid_pad)

    h_all = _experts_fc1(xs, e_w1, be, nvalid)
    slot_out = _experts_fc2(h_all, ws_pad[:, None], e_w2, be, nvalid)
    shared_out = _shared(x, s_w1, s_w2)

    out = _sc_combine(shared_out, slot_out, pos2)
    return out


# speedup vs baseline: 1.1497x; 1.1497x over previous
import functools

import jax
import jax.numpy as jnp
from jax import lax
from jax.experimental import pallas as pl
from jax.experimental.pallas import tpu as pltpu
from jax.experimental.pallas import tpu_sc as plsc

T = 2048
D = 2048
E = 8
K = 2
H = 1408
H2 = 2 * H

B = 128
NB = (K * T) // B + E
S = K * T
S_PAD = NB * B

NC, NS = 2, 16
NW = NC * NS


def _gate_body(x_ref, wg_ref, w_out_ref, e_out_ref):
    sT = lax.dot_general(wg_ref[...], x_ref[...], (((1,), (1,)), ((), ())),
                         preferred_element_type=jnp.float32)
    bt = sT.shape[1]
    m = jnp.max(sT, axis=0, keepdims=True)
    p = jnp.exp(sT - m)
    denom = jnp.sum(p, axis=0, keepdims=True)
    rows = lax.broadcasted_iota(jnp.int32, (E, bt), 0)
    p1 = jnp.max(p, axis=0, keepdims=True)
    e1 = jnp.min(jnp.where(p == p1, rows, E), axis=0, keepdims=True)
    p_m = jnp.where(rows == e1, -jnp.inf, p)
    p2 = jnp.max(p_m, axis=0, keepdims=True)
    e2 = jnp.min(jnp.where(p_m == p2, rows, E), axis=0, keepdims=True)
    w_out_ref[...] = jnp.concatenate([p1, p2], axis=0) / denom
    e_out_ref[...] = jnp.concatenate([e1, e2], axis=0)


def _gate(x, w_gate):
    BT = 512
    return pl.pallas_call(
        _gate_body,
        grid=(T // BT,),
        in_specs=[
            pl.BlockSpec((BT, D), lambda i: (i, 0)),
            pl.BlockSpec((E, D), lambda i: (0, 0)),
        ],
        out_specs=[
            pl.BlockSpec((K, BT), lambda i: (0, i)),
            pl.BlockSpec((K, BT), lambda i: (0, i)),
        ],
        out_shape=[
            jax.ShapeDtypeStruct((K, T), jnp.float32),
            jax.ShapeDtypeStruct((K, T), jnp.int32),
        ],
    )(x, w_gate)


def _routing(e_top, w_top):
    eflat = e_top.reshape(S)
    wflat = w_top.reshape(S)
    order = jnp.argsort(eflat, stable=True)
    sorted_e = eflat[order]
    counts = jnp.bincount(eflat, length=E)
    cum = jnp.concatenate([jnp.zeros(1, jnp.int32),
                           jnp.cumsum(counts)]).astype(jnp.int32)
    nblk = (counts + B - 1) // B
    bcum = jnp.concatenate([jnp.zeros(1, jnp.int32),
                            jnp.cumsum(nblk)]).astype(jnp.int32)
    pad_off = bcum * B
    barange = jnp.arange(NB, dtype=jnp.int32)
    be = jnp.clip(jnp.searchsorted(bcum[1:], barange, side='right'),
                  0, E - 1).astype(jnp.int32)
    nvalid = jnp.clip(counts[be] - (barange - bcum[be]) * B, 0, B)
    j = jnp.arange(S, dtype=jnp.int32)
    padpos = pad_off[sorted_e] + (j - cum[sorted_e])
    tid_pad = jnp.zeros(S_PAD, jnp.int32).at[padpos].set(
        (order % T).astype(jnp.int32))
    ws_pad = jnp.zeros(S_PAD, jnp.float32).at[padpos].set(wflat[order])
    posflat = jnp.zeros(S, jnp.int32).at[order].set(padpos)
    return be, nvalid.astype(jnp.int32), tid_pad, ws_pad, posflat


def _sc_gather(x, tid_pad):
    CH = 16
    NBUF = 3
    RW = S_PAD // NW
    NCH = RW // CH
    mesh = plsc.VectorSubcoreMesh(core_axis_name="c", subcore_axis_name="s")

    @functools.partial(
        pl.kernel,
        out_type=jax.ShapeDtypeStruct((S_PAD, D), jnp.float32),
        mesh=mesh,
        scratch_types=[
            pltpu.VMEM((RW,), jnp.int32),
            [pltpu.VMEM((CH, D), jnp.float32) for _ in range(NBUF)],
            [pltpu.SemaphoreType.DMA for _ in range(NBUF)],
            [pltpu.SemaphoreType.DMA for _ in range(NBUF)],
        ],
    )
    def k(x_hbm, tid_hbm, xs_hbm, idx_v, bufs, gsems, ssems):
        wid = lax.axis_index("s") * NC + lax.axis_index("c")
        base = wid * RW
        pltpu.sync_copy(tid_hbm.at[pl.ds(base, RW)], idx_v)

        def fire(c):
            i = c % NBUF
            return pltpu.async_copy(
                x_hbm.at[idx_v[pl.ds(c * CH, CH)]], bufs[i], gsems[i])

        gd = [fire(c) for c in range(min(NBUF, NCH))]
        sd = [None] * NCH
        for c in range(NCH):
            i = c % NBUF
            gd[c].wait()
            sd[c] = pltpu.async_copy(
                bufs[i], xs_hbm.at[pl.ds(base + c * CH, CH)], ssems[i])
            nxt = c + NBUF
            if nxt < NCH:
                sd[c].wait()
                gd.append(fire(nxt))
        for c in range(max(0, NCH - NBUF), NCH):
            sd[c].wait()

    return k(x, tid_pad)


def _fc1_body(be_ref, nv_ref, xs_ref, w1_ref, h_ref):
    b = pl.program_id(0)

    @pl.when(nv_ref[b] > 0)
    def _():
        y = lax.dot_general(xs_ref[...], w1_ref[0], (((1,), (1,)), ((), ())),
                            preferred_element_type=jnp.float32)
        g = y[:, H:]
        h_ref[...] = y[:, :H] * (g * jax.nn.sigmoid(g))


def _experts_fc1(xs, e_w1, be, nvalid):
    grid_spec = pltpu.PrefetchScalarGridSpec(
        num_scalar_prefetch=2,
        grid=(NB,),
        in_specs=[
            pl.BlockSpec((B, D), lambda b, be, nv: (b, 0)),
            pl.BlockSpec((1, H2, D), lambda b, be, nv: (be[b], 0, 0)),
        ],
        out_specs=pl.BlockSpec((B, H), lambda b, be, nv: (b, 0)),
    )
    return pl.pallas_call(
        _fc1_body,
        grid_spec=grid_spec,
        out_shape=jax.ShapeDtypeStruct((S_PAD, H), jnp.float32),
        compiler_params=pltpu.CompilerParams(
            dimension_semantics=("arbitrary",)),
    )(be, nvalid, xs, e_w1)


def _fc2_body(be_ref, nv_ref, h_ref, ws_ref, w2_ref, out_ref):
    b = pl.program_id(0)

    @pl.when(nv_ref[b] > 0)
    def _():
        z = lax.dot_general(h_ref[...], w2_ref[0], (((1,), (1,)), ((), ())),
                            preferred_element_type=jnp.float32)
        out_ref[...] = z * ws_ref[...]


def _experts_fc2(h_all, ws_col, e_w2, be, nvalid):
    grid_spec = pltpu.PrefetchScalarGridSpec(
        num_scalar_prefetch=2,
        grid=(NB,),
        in_specs=[
            pl.BlockSpec((B, H), lambda b, be, nv: (b, 0)),
            pl.BlockSpec((B, 1), lambda b, be, nv: (b, 0)),
            pl.BlockSpec((1, D, H), lambda b, be, nv: (be[b], 0, 0)),
        ],
        out_specs=pl.BlockSpec((B, D), lambda b, be, nv: (b, 0)),
    )
    return pl.pallas_call(
        _fc2_body,
        grid_spec=grid_spec,
        out_shape=jax.ShapeDtypeStruct((S_PAD, D), jnp.float32),
        compiler_params=pltpu.CompilerParams(
            dimension_semantics=("arbitrary",)),
    )(be, nvalid, h_all, ws_col, e_w2)


def _sfc1_body(x_ref, w1_ref, h_ref):
    y = lax.dot_general(x_ref[...], w1_ref[...], (((1,), (1,)), ((), ())),
                        preferred_element_type=jnp.float32)
    g = y[:, H:]
    h_ref[...] = y[:, :H] * (g * jax.nn.sigmoid(g))


def _sfc2_body(h_ref, routed_ref, w2_ref, out_ref):
    z = lax.dot_general(h_ref[...], w2_ref[...], (((1,), (1,)), ((), ())),
                        preferred_element_type=jnp.float32)
    out_ref[...] = z + routed_ref[...]


def _shared_fc1(x, s_w1):
    BS = 256
    return pl.pallas_call(
        _sfc1_body,
        grid=(T // BS,),
        in_specs=[
            pl.BlockSpec((BS, D), lambda i: (i, 0)),
            pl.BlockSpec((H2, D), lambda i: (0, 0)),
        ],
        out_specs=pl.BlockSpec((BS, H), lambda i: (i, 0)),
        out_shape=jax.ShapeDtypeStruct((T, H), jnp.float32),
    )(x, s_w1)


def _shared_fc2(h_all, routed, s_w2):
    BS = 256
    return pl.pallas_call(
        _sfc2_body,
        grid=(T // BS,),
        in_specs=[
            pl.BlockSpec((BS, H), lambda i: (i, 0)),
            pl.BlockSpec((BS, D), lambda i: (i, 0)),
            pl.BlockSpec((D, H), lambda i: (0, 0)),
        ],
        out_specs=pl.BlockSpec((BS, D), lambda i: (i, 0)),
        out_shape=jax.ShapeDtypeStruct((T, D), jnp.float32),
    )(h_all, routed, s_w2)


def _sc_routed(slot_out, pos2):
    CH = 16
    TW = T // NW
    NCH = TW // CH
    mesh = plsc.VectorSubcoreMesh(core_axis_name="c", subcore_axis_name="s")

    @functools.partial(
        pl.kernel,
        out_type=jax.ShapeDtypeStruct((T, D), jnp.float32),
        mesh=mesh,
        scratch_types=[
            pltpu.VMEM((TW,), jnp.int32),
            pltpu.VMEM((TW,), jnp.int32),
            [pltpu.VMEM((CH, D), jnp.float32) for _ in range(3)],
            [pltpu.SemaphoreType.DMA for _ in range(3)],
            pltpu.SemaphoreType.DMA,
        ],
    )
    def k(sl_hbm, p2_hbm, out_hbm, i0_v, i1_v, bufs, gsems, ssem):
        wid = lax.axis_index("s") * NC + lax.axis_index("c")
        base = wid * TW
        pltpu.sync_copy(p2_hbm.at[pl.ds(base, TW)], i0_v)
        pltpu.sync_copy(p2_hbm.at[pl.ds(T + base, TW)], i1_v)

        def fire(c, j):
            i = (2 * c + j) % 3
            iv = (i0_v, i1_v)[j]
            return pltpu.async_copy(
                sl_hbm.at[iv[pl.ds(c * CH, CH)]], bufs[i], gsems[i])

        gd = {}
        gd[(0, 0)] = fire(0, 0)
        gd[(0, 1)] = fire(0, 1)
        if NCH > 1:
            gd[(1, 0)] = fire(1, 0)
        for c in range(NCH):
            ia = (2 * c) % 3
            ib = (2 * c + 1) % 3
            gd[(c, 0)].wait()
            gd[(c, 1)].wait()
            acc_v, r_v = bufs[ia], bufs[ib]

            def row_add(r, _):
                for ci in range(D // 16):
                    sl = pl.ds(ci * 16, 16)
                    acc_v[r, sl] = acc_v[r, sl] + r_v[r, sl]
                return 0

            lax.fori_loop(0, CH, row_add, 0)
            sdesc = pltpu.async_copy(
                acc_v, out_hbm.at[pl.ds(base + c * CH, CH)], ssem)
            if c + 1 < NCH:
                sdesc.wait()
                gd[(c + 1, 1)] = fire(c + 1, 1)
                if c + 2 < NCH:
                    gd[(c + 2, 0)] = fire(c + 2, 0)
            else:
                sdesc.wait()

    return k(slot_out, pos2)


def kernel(x, w_gate, e_w1, e_w2, s_w1, s_w2):
    w_top, e_top = _gate(x, w_gate)
    be, nvalid, tid_pad, ws_pad, pos2 = _routing(e_top, w_top)

    xs = _sc_gather(x, tid_pad)

    h_sh = _shared_fc1(x, s_w1)
    h_all = _experts_fc1(xs, e_w1, be, nvalid)
    slot_out = _experts_fc2(h_all, ws_pad[:, None], e_w2, be, nvalid)
    routed = _sc_routed(slot_out, pos2)
    out = _shared_fc2(h_sh, routed, s_w2)
    return out
